# baseline (speedup 1.0000x reference)
import functools

import jax
import jax.numpy as jnp
from jax import lax
from jax.experimental import pallas as pl
from jax.experimental.pallas import tpu as pltpu
from jax.experimental.pallas import tpu_sc as plsc

VOCAB = 1_000_000
D = 64
L = 200
OUT_D = 3 * D
LANES = 16
NCHUNK = D // LANES
NFULL = L // LANES
TAIL = L - NFULL * LANES
TOFF = L - LANES
S0, S1 = 128, L - 128
BLKW = 1_024
NPAIR = -(-VOCAB // (2 * BLKW))
PACKED = NPAIR * BLKW


def _repack_body(a_ref, b_ref, o_ref):
    eye = jnp.eye(D, dtype=jnp.float32)
    dn = (((0,), (0,)), ((), ()))
    o_ref[:, 0:D] = lax.dot_general(
        a_ref[...], eye, dn, preferred_element_type=jnp.float32)
    o_ref[:, D:2 * D] = lax.dot_general(
        b_ref[...], eye, dn, preferred_element_type=jnp.float32)


@jax.jit
def _repack(table_t):
    return pl.pallas_call(
        _repack_body,
        grid=(NPAIR,),
        in_specs=[
            pl.BlockSpec((D, BLKW), lambda q: (0, 2 * q)),
            pl.BlockSpec(
                (D, BLKW),
                lambda q: (0, jnp.minimum(2 * q + 1, VOCAB // BLKW))),
        ],
        out_specs=pl.BlockSpec((BLKW, 2 * D), lambda q: (q, 0)),
        out_shape=jax.ShapeDtypeStruct((PACKED, 2 * D), jnp.float32),
    )(table_t, table_t)


@functools.cache
def _build(batch):
    info = plsc.get_sparse_core_info()
    nw = info.num_cores * info.num_subcores
    rpw = batch // nw
    mesh = plsc.VectorSubcoreMesh(core_axis_name="c", subcore_axis_name="s")

    @functools.partial(
        pl.kernel,
        mesh=mesh,
        out_type=jax.ShapeDtypeStruct((batch, OUT_D), jnp.float32),
        scratch_types=[
            pltpu.VMEM((rpw, L), jnp.int32),
            pltpu.VMEM((2, 2, S0), jnp.int32),
            pltpu.VMEM((2, L, 2 * D), jnp.float32),
            pltpu.VMEM((rpw, OUT_D), jnp.float32),
            pltpu.SemaphoreType.DMA,
            pltpu.SemaphoreType.DMA,
        ],
        compiler_params=pltpu.CompilerParams(use_tc_tiling_on_sc=True),
    )
    def disc_kernel(tok_hbm, table_hbm, out_hbm, idx_v, gidx_v, rows_v, out_v,
                    sem0, sem1):
        wid = lax.axis_index("s") * info.num_cores + lax.axis_index("c")
        base = wid * rpw
        pltpu.sync_copy(tok_hbm.at[pl.ds(base, rpw)], idx_v)
        sems = (sem0, sem1)

        def mod_ids(s, r, src_col, dst_h, dst_col):
            v = idx_v[r, pl.ds(src_col, LANES)] & jnp.int32(0x7FFFFFFF)
            row = lax.shift_left(lax.shift_right_logical(v, 11), 10) | (
                v & jnp.int32(BLKW - 1))
            gidx_v[s, dst_h, pl.ds(dst_col, LANES)] = jnp.minimum(
                row, jnp.int32(PACKED - 1))

        def stage_indices(s, r):
            for k in range(NFULL):
                col = k * LANES
                mod_ids(s, r, col, col // S0, col % S0)
            mod_ids(s, r, TOFF, 1, TOFF - S0)

        def gather_descs(s, r):
            return (
                pltpu.make_async_copy(
                    table_hbm.at[gidx_v.at[s, 0]],
                    rows_v.at[s, pl.ds(0, S0)],
                    sems[s]),
                pltpu.make_async_copy(
                    table_hbm.at[gidx_v.at[s, 1, pl.ds(0, S1)]],
                    rows_v.at[s, pl.ds(S0, S1)],
                    sems[s]),
            )

        def start_gather(s, r):
            stage_indices(s, r)
            for dsc in gather_descs(s, r):
                dsc.start()

        def wait_gather(s, r):
            for dsc in gather_descs(s, r):
                dsc.wait()

        def compute(s, r):
            zero = jnp.zeros((LANES,), jnp.float32)
            init = ((zero,) * NCHUNK,) * 5

            def step(l, off, carry):
                a1, a2, a3, ep, t2p = carry
                n1, n2, n3, ne, nt = [], [], [], [], []
                for c in range(NCHUNK):
                    e = rows_v[s, l, pl.ds(off + LANES * c, LANES)]
                    t2 = ep[c] * e
                    n1.append(a1[c] + e)
                    n2.append(a2[c] + t2)
                    n3.append(a3[c] + t2p[c] * e)
                    ne.append(e)
                    nt.append(t2)
                return (tuple(n1), tuple(n2), tuple(n3), tuple(ne), tuple(nt))

            def group(l0, carry, js):
                raw = idx_v[r, pl.ds(l0, LANES)]
                for j in js:
                    off = lax.shift_left(
                        lax.shift_right_logical(raw[j], 10) & jnp.int32(1), 6)
                    carry = step(l0 + j, off, carry)
                return carry

            carry = lax.fori_loop(
                0, NFULL,
                lambda g, cy: group(g * LANES, cy, range(LANES)), init)
            a1, a2, a3, _, _ = group(TOFF, carry, range(LANES - TAIL, LANES))
            for c in range(NCHUNK):
                out_v[r, pl.ds(LANES * c, LANES)] = a1[c] * (1.0 / L)
                out_v[r, pl.ds(D + LANES * c, LANES)] = a2[c] * (1.0 / (L - 1))
                out_v[r, pl.ds(2 * D + LANES * c, LANES)] = (
                    a3[c] * (1.0 / (L - 2)))

        start_gather(0, 0)
        start_gather(1, 1)

        def pair_body(i, carry):
            for s in range(2):
                r = 2 * i + s
                wait_gather(s, r)
                compute(s, r)

                @pl.when(r + 2 < rpw)
                def _():
                    start_gather(s, r + 2)
            return carry

        lax.fori_loop(0, rpw // 2, pair_body, 0)
        pltpu.sync_copy(out_v, out_hbm.at[pl.ds(base, rpw)])

    return disc_kernel


def kernel(token_ids, table):
    tok = token_ids.astype(jnp.int32)
    packed = _repack(jnp.transpose(table))
    return _build(tok.shape[0])(tok, packed)


# speedup vs baseline: 1.4218x; 1.4218x over previous
import functools

import jax
import jax.numpy as jnp
from jax import lax
from jax.experimental import pallas as pl
from jax.experimental.pallas import tpu as pltpu
from jax.experimental.pallas import tpu_sc as plsc

VOCAB = 1_000_000
D = 64
L = 200
OUT_D = 3 * D
LANES = 16
NCHUNK = D // LANES
NFULL = L // LANES
TAIL = L - NFULL * LANES
TOFF = L - LANES
S0, S1 = 128, L - 128
BLKW = 4_096
PBITS = 12
NPAIR = -(-VOCAB // (2 * BLKW))
PACKED = NPAIR * BLKW


def _repack_body(a_ref, b_ref, o_ref):
    eye = jnp.eye(D, dtype=jnp.float32)
    dn = (((0,), (0,)), ((), ()))
    o_ref[:, 0:D] = lax.dot_general(
        a_ref[...], eye, dn, preferred_element_type=jnp.float32)
    o_ref[:, D:2 * D] = lax.dot_general(
        b_ref[...], eye, dn, preferred_element_type=jnp.float32)


@jax.jit
def _repack(table_t):
    return pl.pallas_call(
        _repack_body,
        grid=(NPAIR,),
        in_specs=[
            pl.BlockSpec((D, BLKW), lambda q: (0, 2 * q)),
            pl.BlockSpec(
                (D, BLKW),
                lambda q: (0, jnp.minimum(2 * q + 1, VOCAB // BLKW))),
        ],
        out_specs=pl.BlockSpec((BLKW, 2 * D), lambda q: (q, 0)),
        out_shape=jax.ShapeDtypeStruct((PACKED, 2 * D), jnp.float32),
    )(table_t, table_t)


@functools.cache
def _build(batch):
    info = plsc.get_sparse_core_info()
    nw = info.num_cores * info.num_subcores
    rpw = batch // nw
    mesh = plsc.VectorSubcoreMesh(core_axis_name="c", subcore_axis_name="s")

    @functools.partial(
        pl.kernel,
        mesh=mesh,
        out_type=jax.ShapeDtypeStruct((batch, OUT_D), jnp.float32),
        scratch_types=[
            pltpu.VMEM((rpw, L), jnp.int32),
            pltpu.VMEM((2, 2, S0), jnp.int32),
            pltpu.VMEM((2, L, 2 * D), jnp.float32),
            pltpu.VMEM((rpw, OUT_D), jnp.float32),
            pltpu.SemaphoreType.DMA,
            pltpu.SemaphoreType.DMA,
        ],
        compiler_params=pltpu.CompilerParams(use_tc_tiling_on_sc=True),
    )
    def disc_kernel(tok_hbm, table_hbm, out_hbm, idx_v, gidx_v, rows_v, out_v,
                    sem0, sem1):
        wid = lax.axis_index("s") * info.num_cores + lax.axis_index("c")
        base = wid * rpw
        pltpu.sync_copy(tok_hbm.at[pl.ds(base, rpw)], idx_v)
        sems = (sem0, sem1)

        def mod_ids(s, r, src_col, dst_h, dst_col):
            v = idx_v[r, pl.ds(src_col, LANES)] & jnp.int32(0x7FFFFFFF)
            row = lax.shift_left(
                lax.shift_right_logical(v, PBITS + 1), PBITS) | (
                v & jnp.int32(BLKW - 1))
            gidx_v[s, dst_h, pl.ds(dst_col, LANES)] = jnp.minimum(
                row, jnp.int32(PACKED - 1))

        def stage_indices(s, r):
            for k in range(NFULL):
                col = k * LANES
                mod_ids(s, r, col, col // S0, col % S0)
            mod_ids(s, r, TOFF, 1, TOFF - S0)

        def gather_descs(s, r):
            return (
                pltpu.make_async_copy(
                    table_hbm.at[gidx_v.at[s, 0]],
                    rows_v.at[s, pl.ds(0, S0)],
                    sems[s]),
                pltpu.make_async_copy(
                    table_hbm.at[gidx_v.at[s, 1, pl.ds(0, S1)]],
                    rows_v.at[s, pl.ds(S0, S1)],
                    sems[s]),
            )

        def start_gather(s, r):
            stage_indices(s, r)
            for dsc in gather_descs(s, r):
                dsc.start()

        def wait_gather(s, r):
            for dsc in gather_descs(s, r):
                dsc.wait()

        def compute(s, r):
            zero = jnp.zeros((LANES,), jnp.float32)
            init = ((zero,) * NCHUNK,) * 5

            def step(l, off, carry):
                a1, a2, a3, ep, t2p = carry
                n1, n2, n3, ne, nt = [], [], [], [], []
                for c in range(NCHUNK):
                    e = rows_v[s, l, pl.ds(off + LANES * c, LANES)]
                    t2 = ep[c] * e
                    n1.append(a1[c] + e)
                    n2.append(a2[c] + t2)
                    n3.append(a3[c] + t2p[c] * e)
                    ne.append(e)
                    nt.append(t2)
                return (tuple(n1), tuple(n2), tuple(n3), tuple(ne), tuple(nt))

            def group(l0, carry, js):
                raw = idx_v[r, pl.ds(l0, LANES)]
                for j in js:
                    off = lax.shift_left(
                        lax.shift_right_logical(raw[j], PBITS)
                        & jnp.int32(1), 6)
                    carry = step(l0 + j, off, carry)
                return carry

            carry = lax.fori_loop(
                0, NFULL,
                lambda g, cy: group(g * LANES, cy, range(LANES)), init)
            a1, a2, a3, _, _ = group(TOFF, carry, range(LANES - TAIL, LANES))
            for c in range(NCHUNK):
                out_v[r, pl.ds(LANES * c, LANES)] = a1[c] * (1.0 / L)
                out_v[r, pl.ds(D + LANES * c, LANES)] = a2[c] * (1.0 / (L - 1))
                out_v[r, pl.ds(2 * D + LANES * c, LANES)] = (
                    a3[c] * (1.0 / (L - 2)))

        start_gather(0, 0)
        start_gather(1, 1)

        def pair_body(i, carry):
            for s in range(2):
                r = 2 * i + s
                wait_gather(s, r)
                compute(s, r)

                @pl.when(r + 2 < rpw)
                def _():
                    start_gather(s, r + 2)
            return carry

        lax.fori_loop(0, rpw // 2, pair_body, 0)
        pltpu.sync_copy(out_v, out_hbm.at[pl.ds(base, rpw)])

    return disc_kernel


def kernel(token_ids, table):
    tok = token_ids.astype(jnp.int32)
    packed = _repack(jnp.transpose(table))
    return _build(tok.shape[0])(tok, packed)


# speedup vs baseline: 1.5265x; 1.0737x over previous
import functools

import jax
import jax.numpy as jnp
from jax import lax
from jax.experimental import pallas as pl
from jax.experimental.pallas import tpu as pltpu
from jax.experimental.pallas import tpu_sc as plsc

VOCAB = 1_000_000
D = 64
L = 200
OUT_D = 3 * D
LANES = 16
NCHUNK = D // LANES
NFULL = L // LANES
TAIL = L - NFULL * LANES
TOFF = L - LANES
S0, S1 = 128, L - 128
BLKW = 8_192
PBITS = 13
NPAIR = -(-VOCAB // (2 * BLKW))
PACKED = NPAIR * BLKW


def _repack_body(a_ref, b_ref, o_ref):
    eye = jnp.eye(D, dtype=jnp.float32)
    dn = (((0,), (0,)), ((), ()))
    o_ref[:, 0:D] = lax.dot_general(
        a_ref[...], eye, dn, preferred_element_type=jnp.float32)
    o_ref[:, D:2 * D] = lax.dot_general(
        b_ref[...], eye, dn, preferred_element_type=jnp.float32)


@jax.jit
def _repack(table_t):
    return pl.pallas_call(
        _repack_body,
        grid=(NPAIR,),
        in_specs=[
            pl.BlockSpec((D, BLKW), lambda q: (0, 2 * q)),
            pl.BlockSpec(
                (D, BLKW),
                lambda q: (0, jnp.minimum(2 * q + 1, VOCAB // BLKW))),
        ],
        out_specs=pl.BlockSpec((BLKW, 2 * D), lambda q: (q, 0)),
        out_shape=jax.ShapeDtypeStruct((PACKED, 2 * D), jnp.float32),
    )(table_t, table_t)


@functools.cache
def _build(batch):
    info = plsc.get_sparse_core_info()
    nw = info.num_cores * info.num_subcores
    rpw = batch // nw
    mesh = plsc.VectorSubcoreMesh(core_axis_name="c", subcore_axis_name="s")

    @functools.partial(
        pl.kernel,
        mesh=mesh,
        out_type=jax.ShapeDtypeStruct((batch, OUT_D), jnp.float32),
        scratch_types=[
            pltpu.VMEM((rpw, L), jnp.int32),
            pltpu.VMEM((2, 2, S0), jnp.int32),
            pltpu.VMEM((2, L, 2 * D), jnp.float32),
            pltpu.VMEM((rpw, OUT_D), jnp.float32),
            pltpu.SemaphoreType.DMA,
            pltpu.SemaphoreType.DMA,
        ],
        compiler_params=pltpu.CompilerParams(use_tc_tiling_on_sc=True),
    )
    def disc_kernel(tok_hbm, table_hbm, out_hbm, idx_v, gidx_v, rows_v, out_v,
                    sem0, sem1):
        wid = lax.axis_index("s") * info.num_cores + lax.axis_index("c")
        base = wid * rpw
        pltpu.sync_copy(tok_hbm.at[pl.ds(base, rpw)], idx_v)
        sems = (sem0, sem1)

        def mod_ids(s, r, src_col, dst_h, dst_col):
            v = idx_v[r, pl.ds(src_col, LANES)] & jnp.int32(0x7FFFFFFF)
            row = lax.shift_left(
                lax.shift_right_logical(v, PBITS + 1), PBITS) | (
                v & jnp.int32(BLKW - 1))
            gidx_v[s, dst_h, pl.ds(dst_col, LANES)] = jnp.minimum(
                row, jnp.int32(PACKED - 1))

        def stage_indices(s, r):
            for k in range(NFULL):
                col = k * LANES
                mod_ids(s, r, col, col // S0, col % S0)
            mod_ids(s, r, TOFF, 1, TOFF - S0)

        def gather_descs(s, r):
            return (
                pltpu.make_async_copy(
                    table_hbm.at[gidx_v.at[s, 0]],
                    rows_v.at[s, pl.ds(0, S0)],
                    sems[s]),
                pltpu.make_async_copy(
                    table_hbm.at[gidx_v.at[s, 1, pl.ds(0, S1)]],
                    rows_v.at[s, pl.ds(S0, S1)],
                    sems[s]),
            )

        def start_gather(s, r):
            stage_indices(s, r)
            for dsc in gather_descs(s, r):
                dsc.start()

        def wait_gather(s, r):
            for dsc in gather_descs(s, r):
                dsc.wait()

        def compute(s, r):
            zero = jnp.zeros((LANES,), jnp.float32)
            init = ((zero,) * NCHUNK,) * 5

            def step(l, off, carry):
                a1, a2, a3, ep, t2p = carry
                n1, n2, n3, ne, nt = [], [], [], [], []
                for c in range(NCHUNK):
                    e = rows_v[s, l, pl.ds(off + LANES * c, LANES)]
                    t2 = ep[c] * e
                    n1.append(a1[c] + e)
                    n2.append(a2[c] + t2)
                    n3.append(a3[c] + t2p[c] * e)
                    ne.append(e)
                    nt.append(t2)
                return (tuple(n1), tuple(n2), tuple(n3), tuple(ne), tuple(nt))

            def group(l0, carry, js):
                raw = idx_v[r, pl.ds(l0, LANES)]
                for j in js:
                    off = lax.shift_left(
                        lax.shift_right_logical(raw[j], PBITS)
                        & jnp.int32(1), 6)
                    carry = step(l0 + j, off, carry)
                return carry

            carry = lax.fori_loop(
                0, NFULL,
                lambda g, cy: group(g * LANES, cy, range(LANES)), init)
            a1, a2, a3, _, _ = group(TOFF, carry, range(LANES - TAIL, LANES))
            for c in range(NCHUNK):
                out_v[r, pl.ds(LANES * c, LANES)] = a1[c] * (1.0 / L)
                out_v[r, pl.ds(D + LANES * c, LANES)] = a2[c] * (1.0 / (L - 1))
                out_v[r, pl.ds(2 * D + LANES * c, LANES)] = (
                    a3[c] * (1.0 / (L - 2)))

        start_gather(0, 0)
        start_gather(1, 1)

        def pair_body(i, carry):
            for s in range(2):
                r = 2 * i + s
                wait_gather(s, r)
                compute(s, r)

                @pl.when(r + 2 < rpw)
                def _():
                    start_gather(s, r + 2)
            return carry

        lax.fori_loop(0, rpw // 2, pair_body, 0)
        pltpu.sync_copy(out_v, out_hbm.at[pl.ds(base, rpw)])

    return disc_kernel


def kernel(token_ids, table):
    tok = token_ids.astype(jnp.int32)
    packed = _repack(jnp.transpose(table))
    return _build(tok.shape[0])(tok, packed)


# speedup vs baseline: 1.5814x; 1.0359x over previous
import functools

import jax
import jax.numpy as jnp
from jax import lax
from jax.experimental import pallas as pl
from jax.experimental.pallas import tpu as pltpu
from jax.experimental.pallas import tpu_sc as plsc

VOCAB = 1_000_000
D = 64
L = 200
OUT_D = 3 * D
LANES = 16
NCHUNK = D // LANES
NFULL = L // LANES
TAIL = L - NFULL * LANES
TOFF = L - LANES
S0, S1 = 128, L - 128
BLKW = 16_384
PBITS = 14
NPAIR = -(-VOCAB // (2 * BLKW))
PACKED = NPAIR * BLKW


def _repack_body(a_ref, b_ref, o_ref):
    eye = jnp.eye(D, dtype=jnp.float32)
    dn = (((0,), (0,)), ((), ()))
    o_ref[:, 0:D] = lax.dot_general(
        a_ref[...], eye, dn, preferred_element_type=jnp.float32)
    o_ref[:, D:2 * D] = lax.dot_general(
        b_ref[...], eye, dn, preferred_element_type=jnp.float32)


@jax.jit
def _repack(table_t):
    return pl.pallas_call(
        _repack_body,
        grid=(NPAIR,),
        in_specs=[
            pl.BlockSpec((D, BLKW), lambda q: (0, 2 * q)),
            pl.BlockSpec(
                (D, BLKW),
                lambda q: (0, jnp.minimum(2 * q + 1, VOCAB // BLKW))),
        ],
        out_specs=pl.BlockSpec((BLKW, 2 * D), lambda q: (q, 0)),
        out_shape=jax.ShapeDtypeStruct((PACKED, 2 * D), jnp.float32),
    )(table_t, table_t)


@functools.cache
def _build(batch):
    info = plsc.get_sparse_core_info()
    nw = info.num_cores * info.num_subcores
    rpw = batch // nw
    mesh = plsc.VectorSubcoreMesh(core_axis_name="c", subcore_axis_name="s")

    @functools.partial(
        pl.kernel,
        mesh=mesh,
        out_type=jax.ShapeDtypeStruct((batch, OUT_D), jnp.float32),
        scratch_types=[
            pltpu.VMEM((rpw, L), jnp.int32),
            pltpu.VMEM((2, 2, S0), jnp.int32),
            pltpu.VMEM((2, L, 2 * D), jnp.float32),
            pltpu.VMEM((rpw, OUT_D), jnp.float32),
            pltpu.SemaphoreType.DMA,
            pltpu.SemaphoreType.DMA,
        ],
        compiler_params=pltpu.CompilerParams(use_tc_tiling_on_sc=True),
    )
    def disc_kernel(tok_hbm, table_hbm, out_hbm, idx_v, gidx_v, rows_v, out_v,
                    sem0, sem1):
        wid = lax.axis_index("s") * info.num_cores + lax.axis_index("c")
        base = wid * rpw
        pltpu.sync_copy(tok_hbm.at[pl.ds(base, rpw)], idx_v)
        sems = (sem0, sem1)

        def mod_ids(s, r, src_col, dst_h, dst_col):
            v = idx_v[r, pl.ds(src_col, LANES)] & jnp.int32(0x7FFFFFFF)
            row = lax.shift_left(
                lax.shift_right_logical(v, PBITS + 1), PBITS) | (
                v & jnp.int32(BLKW - 1))
            gidx_v[s, dst_h, pl.ds(dst_col, LANES)] = jnp.minimum(
                row, jnp.int32(PACKED - 1))

        def stage_indices(s, r):
            for k in range(NFULL):
                col = k * LANES
                mod_ids(s, r, col, col // S0, col % S0)
            mod_ids(s, r, TOFF, 1, TOFF - S0)

        def gather_descs(s, r):
            return (
                pltpu.make_async_copy(
                    table_hbm.at[gidx_v.at[s, 0]],
                    rows_v.at[s, pl.ds(0, S0)],
                    sems[s]),
                pltpu.make_async_copy(
                    table_hbm.at[gidx_v.at[s, 1, pl.ds(0, S1)]],
                    rows_v.at[s, pl.ds(S0, S1)],
                    sems[s]),
            )

        def start_gather(s, r):
            stage_indices(s, r)
            for dsc in gather_descs(s, r):
                dsc.start()

        def wait_gather(s, r):
            for dsc in gather_descs(s, r):
                dsc.wait()

        def compute(s, r):
            zero = jnp.zeros((LANES,), jnp.float32)
            init = ((zero,) * NCHUNK,) * 5

            def step(l, off, carry):
                a1, a2, a3, ep, t2p = carry
                n1, n2, n3, ne, nt = [], [], [], [], []
                for c in range(NCHUNK):
                    e = rows_v[s, l, pl.ds(off + LANES * c, LANES)]
                    t2 = ep[c] * e
                    n1.append(a1[c] + e)
                    n2.append(a2[c] + t2)
                    n3.append(a3[c] + t2p[c] * e)
                    ne.append(e)
                    nt.append(t2)
                return (tuple(n1), tuple(n2), tuple(n3), tuple(ne), tuple(nt))

            def group(l0, carry, js):
                raw = idx_v[r, pl.ds(l0, LANES)]
                for j in js:
                    off = lax.shift_left(
                        lax.shift_right_logical(raw[j], PBITS)
                        & jnp.int32(1), 6)
                    carry = step(l0 + j, off, carry)
                return carry

            carry = lax.fori_loop(
                0, NFULL,
                lambda g, cy: group(g * LANES, cy, range(LANES)), init)
            a1, a2, a3, _, _ = group(TOFF, carry, range(LANES - TAIL, LANES))
            for c in range(NCHUNK):
                out_v[r, pl.ds(LANES * c, LANES)] = a1[c] * (1.0 / L)
                out_v[r, pl.ds(D + LANES * c, LANES)] = a2[c] * (1.0 / (L - 1))
                out_v[r, pl.ds(2 * D + LANES * c, LANES)] = (
                    a3[c] * (1.0 / (L - 2)))

        start_gather(0, 0)
        start_gather(1, 1)

        def pair_body(i, carry):
            for s in range(2):
                r = 2 * i + s
                wait_gather(s, r)
                compute(s, r)

                @pl.when(r + 2 < rpw)
                def _():
                    start_gather(s, r + 2)
            return carry

        lax.fori_loop(0, rpw // 2, pair_body, 0)
        pltpu.sync_copy(out_v, out_hbm.at[pl.ds(base, rpw)])

    return disc_kernel


def kernel(token_ids, table):
    tok = token_ids.astype(jnp.int32)
    packed = _repack(jnp.transpose(table))
    return _build(tok.shape[0])(tok, packed)


# speedup vs baseline: 1.7826x; 1.1272x over previous
import functools

import jax
import jax.numpy as jnp
from jax import lax
from jax.experimental import pallas as pl
from jax.experimental.pallas import tpu as pltpu
from jax.experimental.pallas import tpu_sc as plsc

VOCAB = 1_000_000
D = 64
L = 200
OUT_D = 3 * D
LANES = 16
NCHUNK = D // LANES
NFULL = L // LANES
TAIL = L - NFULL * LANES
TOFF = L - LANES
S0, S1 = 128, L - 128
BLKW = 16_384
PBITS = 14
NPAIR = -(-VOCAB // (2 * BLKW))
PACKED = NPAIR * BLKW


def _repack_body(a_ref, b_ref, o_ref):
    eye = jnp.eye(2 * D, dtype=jnp.float32)
    dn = (((0,), (0,)), ((), ()))
    c = jnp.concatenate([a_ref[...], b_ref[...]], axis=0)
    o_ref[...] = lax.dot_general(
        c, eye, dn, preferred_element_type=jnp.float32)


@jax.jit
def _repack(table_t):
    return pl.pallas_call(
        _repack_body,
        grid=(NPAIR,),
        in_specs=[
            pl.BlockSpec((D, BLKW), lambda q: (0, 2 * q)),
            pl.BlockSpec(
                (D, BLKW),
                lambda q: (0, jnp.minimum(2 * q + 1, VOCAB // BLKW))),
        ],
        out_specs=pl.BlockSpec((BLKW, 2 * D), lambda q: (q, 0)),
        out_shape=jax.ShapeDtypeStruct((PACKED, 2 * D), jnp.float32),
    )(table_t, table_t)


@functools.cache
def _build(batch):
    info = plsc.get_sparse_core_info()
    nw = info.num_cores * info.num_subcores
    rpw = batch // nw
    mesh = plsc.VectorSubcoreMesh(core_axis_name="c", subcore_axis_name="s")

    @functools.partial(
        pl.kernel,
        mesh=mesh,
        out_type=jax.ShapeDtypeStruct((batch, OUT_D), jnp.float32),
        scratch_types=[
            pltpu.VMEM((rpw, L), jnp.int32),
            pltpu.VMEM((2, 2, S0), jnp.int32),
            pltpu.VMEM((2, L, 2 * D), jnp.float32),
            pltpu.VMEM((rpw, OUT_D), jnp.float32),
            pltpu.SemaphoreType.DMA,
            pltpu.SemaphoreType.DMA,
        ],
        compiler_params=pltpu.CompilerParams(use_tc_tiling_on_sc=True),
    )
    def disc_kernel(tok_hbm, table_hbm, out_hbm, idx_v, gidx_v, rows_v, out_v,
                    sem0, sem1):
        wid = lax.axis_index("s") * info.num_cores + lax.axis_index("c")
        base = wid * rpw
        pltpu.sync_copy(tok_hbm.at[pl.ds(base, rpw)], idx_v)
        sems = (sem0, sem1)

        def mod_ids(s, r, src_col, dst_h, dst_col):
            v = idx_v[r, pl.ds(src_col, LANES)] & jnp.int32(0x7FFFFFFF)
            row = lax.shift_left(
                lax.shift_right_logical(v, PBITS + 1), PBITS) | (
                v & jnp.int32(BLKW - 1))
            gidx_v[s, dst_h, pl.ds(dst_col, LANES)] = jnp.minimum(
                row, jnp.int32(PACKED - 1))

        def stage_indices(s, r):
            for k in range(NFULL):
                col = k * LANES
                mod_ids(s, r, col, col // S0, col % S0)
            mod_ids(s, r, TOFF, 1, TOFF - S0)

        def gather_descs(s, r):
            return (
                pltpu.make_async_copy(
                    table_hbm.at[gidx_v.at[s, 0]],
                    rows_v.at[s, pl.ds(0, S0)],
                    sems[s]),
                pltpu.make_async_copy(
                    table_hbm.at[gidx_v.at[s, 1, pl.ds(0, S1)]],
                    rows_v.at[s, pl.ds(S0, S1)],
                    sems[s]),
            )

        def start_gather(s, r):
            stage_indices(s, r)
            for dsc in gather_descs(s, r):
                dsc.start()

        def wait_gather(s, r):
            for dsc in gather_descs(s, r):
                dsc.wait()

        def compute(s, r):
            zero = jnp.zeros((LANES,), jnp.float32)
            init = ((zero,) * NCHUNK,) * 5

            def step(l, off, carry):
                a1, a2, a3, ep, t2p = carry
                n1, n2, n3, ne, nt = [], [], [], [], []
                for c in range(NCHUNK):
                    e = rows_v[s, l, pl.ds(off + LANES * c, LANES)]
                    t2 = ep[c] * e
                    n1.append(a1[c] + e)
                    n2.append(a2[c] + t2)
                    n3.append(a3[c] + t2p[c] * e)
                    ne.append(e)
                    nt.append(t2)
                return (tuple(n1), tuple(n2), tuple(n3), tuple(ne), tuple(nt))

            def group(l0, carry, js):
                raw = idx_v[r, pl.ds(l0, LANES)]
                for j in js:
                    off = lax.shift_left(
                        lax.shift_right_logical(raw[j], PBITS)
                        & jnp.int32(1), 6)
                    carry = step(l0 + j, off, carry)
                return carry

            carry = lax.fori_loop(
                0, NFULL,
                lambda g, cy: group(g * LANES, cy, range(LANES)), init)
            a1, a2, a3, _, _ = group(TOFF, carry, range(LANES - TAIL, LANES))
            for c in range(NCHUNK):
                out_v[r, pl.ds(LANES * c, LANES)] = a1[c] * (1.0 / L)
                out_v[r, pl.ds(D + LANES * c, LANES)] = a2[c] * (1.0 / (L - 1))
                out_v[r, pl.ds(2 * D + LANES * c, LANES)] = (
                    a3[c] * (1.0 / (L - 2)))

        start_gather(0, 0)
        start_gather(1, 1)

        def pair_body(i, carry):
            for s in range(2):
                r = 2 * i + s
                wait_gather(s, r)
                compute(s, r)

                @pl.when(r + 2 < rpw)
                def _():
                    start_gather(s, r + 2)
            return carry

        lax.fori_loop(0, rpw // 2, pair_body, 0)
        pltpu.sync_copy(out_v, out_hbm.at[pl.ds(base, rpw)])

    return disc_kernel


def kernel(token_ids, table):
    tok = token_ids.astype(jnp.int32)
    packed = _repack(jnp.transpose(table))
    return _build(tok.shape[0])(tok, packed)
